# Initial kernel scaffold; baseline (speedup 1.0000x reference)
import functools

import jax
import jax.numpy as jnp
from jax import lax
from jax.experimental import pallas as pl
from jax.experimental.pallas import tpu as pltpu
from jax.experimental.pallas import tpu_sc as plsc

DIM = 32
NC, NS = 2, 16
NW = NC * NS
CHUNK = 1024
IDX_W = 128
IDX_ROWS = CHUNK // IDX_W


@functools.lru_cache(maxsize=None)
def _build(n_total):
    rows_per_w = n_total // NW
    steps = rows_per_w // CHUNK
    irows_per_w = rows_per_w // IDX_W
    mesh = plsc.VectorSubcoreMesh(core_axis_name="c", subcore_axis_name="s")

    def body(x_hbm, table_hbm, out_hbm, idx_v, rows_v, gsem):
        wid = lax.axis_index("s") * NC + lax.axis_index("c")
        row_base = wid * rows_per_w
        irow_base = wid * irows_per_w

        def step_fn(g, carry):
            pltpu.sync_copy(
                x_hbm.at[pl.ds(irow_base + g * IDX_ROWS, IDX_ROWS)], idx_v)
            descs = []
            for j in range(IDX_ROWS):
                descs.append(pltpu.async_copy(
                    table_hbm.at[idx_v.at[j]],
                    rows_v.at[pl.ds(j * IDX_W, IDX_W)],
                    gsem))
            for d in descs:
                d.wait()
            pltpu.sync_copy(
                rows_v, out_hbm.at[pl.ds(row_base + g * CHUNK, CHUNK)])
            return carry

        lax.fori_loop(0, steps, step_fn, 0)

    return pl.kernel(
        body,
        out_type=jax.ShapeDtypeStruct((n_total, DIM), jnp.float32),
        mesh=mesh,
        scratch_types=[
            pltpu.VMEM((IDX_ROWS, IDX_W), jnp.int32),
            pltpu.VMEM((CHUNK, DIM), jnp.float32),
            pltpu.SemaphoreType.DMA,
        ],
    )


@jax.jit
def kernel(x, table):
    b, s = x.shape
    n = b * s
    x2 = x.reshape(n // IDX_W, IDX_W).astype(jnp.int32)
    out = _build(n)(x2, table)
    return out.reshape(b, s, DIM)


# speedup vs baseline: 1.4584x; 1.4584x over previous
import functools

import jax
import jax.numpy as jnp
from jax import lax
from jax.experimental import pallas as pl
from jax.experimental.pallas import tpu as pltpu
from jax.experimental.pallas import tpu_sc as plsc

DIM = 32
NC, NS = 2, 16
NW = NC * NS
CHUNK = 1024
IDX_W = 128
IDX_ROWS = CHUNK // IDX_W


@functools.lru_cache(maxsize=None)
def _build(n_total):
    rows_per_w = n_total // NW
    steps = rows_per_w // CHUNK
    irows_per_w = rows_per_w // IDX_W
    mesh = plsc.VectorSubcoreMesh(core_axis_name="c", subcore_axis_name="s")

    def body(x_hbm, table_hbm, out_hbm, idx_v, rows_v, gsem):
        wid = lax.axis_index("s") * NC + lax.axis_index("c")
        row_base = wid * rows_per_w
        irow_base = wid * irows_per_w

        def step_fn(g, carry):
            pltpu.sync_copy(
                x_hbm.at[pl.ds(irow_base + g * IDX_ROWS, IDX_ROWS)], idx_v)
            descs = []
            for j in range(IDX_ROWS):
                descs.append(pltpu.async_copy(
                    table_hbm.at[idx_v.at[j]],
                    rows_v.at[pl.ds(j * IDX_W, IDX_W)],
                    gsem))
            for d in descs:
                d.wait()
            pltpu.sync_copy(
                rows_v, out_hbm.at[pl.ds(row_base + g * CHUNK, CHUNK)])
            return carry

        lax.fori_loop(0, steps, step_fn, 0)

    return pl.kernel(
        body,
        out_type=jax.ShapeDtypeStruct((n_total, DIM), jnp.float32),
        mesh=mesh,
        scratch_types=[
            pltpu.VMEM((IDX_ROWS, IDX_W), jnp.int32),
            pltpu.VMEM((CHUNK, DIM), jnp.float32),
            pltpu.SemaphoreType.DMA,
        ],
        compiler_params=pltpu.CompilerParams(use_tc_tiling_on_sc=False),
    )


@jax.jit
def kernel(x, table):
    b, s = x.shape
    n = b * s
    x2 = x.reshape(n // IDX_W, IDX_W).astype(jnp.int32)
    out = _build(n)(x2, table)
    return out.reshape(b, s, DIM)


# speedup vs baseline: 1.4998x; 1.0284x over previous
import functools

import jax
import jax.numpy as jnp
from jax import lax
from jax.experimental import pallas as pl
from jax.experimental.pallas import tpu as pltpu
from jax.experimental.pallas import tpu_sc as plsc

DIM = 32
NC, NS = 2, 16
NW = NC * NS
IDX_W = 128
CHUNK = 1280
NBUF = 2


@functools.lru_cache(maxsize=None)
def _build(n_total, chunk, nbuf):
    rows_per_w = n_total // NW
    steps = rows_per_w // chunk
    assert steps * chunk == rows_per_w
    assert (steps - nbuf) % nbuf == 0 and steps >= 2 * nbuf
    irows = chunk // IDX_W
    tot_irows = rows_per_w // IDX_W
    mesh = plsc.VectorSubcoreMesh(core_axis_name="c", subcore_axis_name="s")

    def body(x_hbm, table_hbm, out_hbm, idx_all, *scratch):
        bufs = scratch[:nbuf]
        gsems = scratch[nbuf:2 * nbuf]
        osems = scratch[2 * nbuf:3 * nbuf]
        wid = lax.axis_index("s") * NC + lax.axis_index("c")
        row_base = wid * rows_per_w
        irow_base = wid * tot_irows

        pltpu.sync_copy(x_hbm.at[pl.ds(irow_base, tot_irows)], idx_all)

        def fire_gather(c, b):
            for j in range(irows):
                pltpu.async_copy(
                    table_hbm.at[idx_all.at[c * irows + j]],
                    bufs[b].at[pl.ds(j * IDX_W, IDX_W)],
                    gsems[b])

        def drain_gather(c, b):
            for j in range(irows):
                pltpu.make_async_copy(
                    table_hbm.at[idx_all.at[c * irows + j]],
                    bufs[b].at[pl.ds(j * IDX_W, IDX_W)],
                    gsems[b]).wait()

        def out_copy(c, b):
            return pltpu.async_copy(
                bufs[b], out_hbm.at[pl.ds(row_base + c * chunk, chunk)],
                osems[b])

        def drain_out(c, b):
            pltpu.make_async_copy(
                bufs[b], out_hbm.at[pl.ds(row_base + c * chunk, chunk)],
                osems[b]).wait()

        for b in range(nbuf):
            fire_gather(b, b)

        def step_fn(it, carry):
            for b in range(nbuf):
                c = it * nbuf + b
                drain_gather(c, b)
                out_copy(c, b)
                drain_out(c, b)
                fire_gather(c + nbuf, b)
            return carry

        lax.fori_loop(0, (steps - nbuf) // nbuf, step_fn, 0)

        for b in range(nbuf):
            c = steps - nbuf + b
            drain_gather(c, b)
            out_copy(c, b)
            drain_out(c, b)

    return pl.kernel(
        body,
        out_type=jax.ShapeDtypeStruct((n_total, DIM), jnp.float32),
        mesh=mesh,
        scratch_types=(
            [pltpu.VMEM((tot_irows, IDX_W), jnp.int32)]
            + [pltpu.VMEM((chunk, DIM), jnp.float32) for _ in range(nbuf)]
            + [pltpu.SemaphoreType.DMA for _ in range(2 * nbuf)]
        ),
        compiler_params=pltpu.CompilerParams(use_tc_tiling_on_sc=False),
    )


@jax.jit
def kernel(x, table):
    b, s = x.shape
    n = b * s
    x2 = x.reshape(n // IDX_W, IDX_W).astype(jnp.int32)
    out = _build(n, CHUNK, NBUF)(x2, table)
    return out.reshape(b, s, DIM)
